# baseline (speedup 1.0000x reference)
import jax
import jax.numpy as jnp
from jax.experimental import pallas as pl
from jax.experimental.pallas import tpu as pltpu

_STEP_CURRENT = 10
_N_HIST = _STEP_CURRENT + 1
_N_TARGET = 8
_PI = 3.141592653589793
_HI = jax.lax.Precision.HIGHEST
_LO = jax.lax.Precision.DEFAULT
_SCENES_PER_STEP = 32
_T = 91
_A = 64


def _wrap_rad(x):
    m = x + _PI
    m = m - (2.0 * _PI) * jnp.floor(m / (2.0 * _PI))
    return m - _PI


def _dot_t(a, b, prec):
    return jax.lax.dot_general(
        a, b, (((1,), (1,)), ((), ())), precision=prec,
        preferred_element_type=jnp.float32)


def _dot(a, b, prec):
    return jax.lax.dot_general(
        a, b, (((1,), (0,)), ((), ())), precision=prec,
        preferred_element_type=jnp.float32)


def _group_kernel(pos_ref, vel_ref, spd_ref, acc_ref, yawr_ref, valid_ref,
                  yaw_ref, statics_ref,
                  o_idx, o_refpos, o_refrot, o_type, o_role,
                  o_tvalid, o_tpos, o_tvel, o_tspd, o_tacc, o_tyaw, o_tyawr,
                  o_size, o_gvalid, o_gpos, o_gspd, o_gvel, o_gyaw, o_gcmd):
    A = _A
    P = _N_TARGET
    T = _T
    H = _N_HIST
    F = T - H
    C = _STEP_CURRENT
    G = _SCENES_PER_STEP

    l_row = jax.lax.broadcasted_iota(jnp.int32, (A, 2 * A), 1)
    a2_col = 2 * jax.lax.broadcasted_iota(jnp.int32, (A, 2 * A), 0)
    d0 = (l_row == a2_col).astype(jnp.float32)
    d1 = (l_row == a2_col + 1).astype(jnp.float32)
    t_row_h = jax.lax.broadcasted_iota(jnp.int32, (H, 2 * H), 1)
    t2_col_h = 2 * jax.lax.broadcasted_iota(jnp.int32, (H, 2 * H), 0)
    e0h = (t_row_h == t2_col_h).astype(jnp.float32)
    e1h = (t_row_h == t2_col_h + 1).astype(jnp.float32)
    t_row_f = jax.lax.broadcasted_iota(jnp.int32, (F, 2 * F), 1)
    t2_col_f = 2 * jax.lax.broadcasted_iota(jnp.int32, (F, 2 * F), 0)
    e0f = (t_row_f == t2_col_f).astype(jnp.float32)
    e1f = (t_row_f == t2_col_f + 1).astype(jnp.float32)
    p_col = jax.lax.broadcasted_iota(jnp.int32, (P, 1), 0)
    a_row = jax.lax.broadcasted_iota(jnp.int32, (P, A), 1)
    neg_a_row1 = A - 1 - jax.lax.broadcasted_iota(jnp.int32, (1, A), 1)

    sel_fs, sel2xs, sel2ys = [], [], []
    for g in range(G):
        role = statics_ref[g, :, 3:6]
        w_col = jnp.sum(role, axis=1, keepdims=True)
        w_row = jnp.transpose(w_col) + valid_ref[g, C:C + 1, :].astype(
            jnp.float32)
        key_row = w_row.astype(jnp.int32) * A + neg_a_row1
        key_col = jnp.transpose(key_row)
        rank_col = jnp.sum((key_row > key_col).astype(jnp.int32), axis=1,
                           keepdims=True)
        rank_row = jnp.transpose(rank_col)
        sel = (rank_row == p_col)
        idx_col = jnp.sum(jnp.where(sel, a_row, 0), axis=1, keepdims=True)
        o_idx[g] = jnp.transpose(idx_col)
        sel_f = sel.astype(jnp.float32)
        sel_fs.append(sel_f)
        sel2xs.append(_dot(sel_f, d0, _LO))
        sel2ys.append(_dot(sel_f, d1, _LO))

    gath = []
    for g in range(G):
        px = _dot_t(sel2xs[g], pos_ref[g], _HI)
        py = _dot_t(sel2ys[g], pos_ref[g], _HI)
        vx = _dot_t(sel2xs[g], vel_ref[g], _HI)
        vy = _dot_t(sel2ys[g], vel_ref[g], _HI)
        g_spd = _dot_t(sel_fs[g], spd_ref[g], _LO)
        g_acc = _dot_t(sel_fs[g], acc_ref[g], _LO)
        g_yawr = _dot_t(sel_fs[g], yawr_ref[g], _LO)
        g_valid = _dot_t(sel_fs[g], valid_ref[g].astype(jnp.float32), _LO)
        g_yaw = _dot_t(sel_fs[g], yaw_ref[g], _HI)
        st = _dot(sel_fs[g], statics_ref[g], _LO)
        gath.append((px, py, vx, vy, g_spd, g_acc, g_yawr, g_valid,
                     g_yaw, st))

    for g in range(G):
        px, py, vx, vy, g_spd, g_acc, g_yawr, g_valid, g_yaw, st = gath[g]

        px0 = px[:, C:C + 1]
        py0 = py[:, C:C + 1]
        yaw0 = g_yaw[:, C:C + 1]
        c = jnp.cos(yaw0)
        s = jnp.sin(yaw0)

        dx = px - px0
        dy = py - py0
        lx = dx * c + dy * s
        ly = dy * c - dx * s
        lvx = vx * c + vy * s
        lvy = vy * c - vx * s
        lyaw = _wrap_rad(g_yaw - yaw0)

        o_refpos[g] = jnp.concatenate([px0, py0], axis=1)
        o_refrot[g] = jnp.concatenate([c, -s, s, c], axis=1)
        o_type[g] = st[:, 0:3] > 0.5
        o_role[g] = st[:, 3:6] > 0.5
        o_size[g] = st[:, 6:9]
        o_gcmd[g] = st[:, 9:17]

        xh = jnp.concatenate([lx[:, :H], lvx[:, :H]], axis=0)
        yh = jnp.concatenate([ly[:, :H], lvy[:, :H]], axis=0)
        rh = _dot(xh, e0h, _LO) + _dot(yh, e1h, _LO)
        o_tpos[g] = rh[:P]
        o_tvel[g] = rh[P:]
        xf = jnp.concatenate([lx[:, H:], lvx[:, H:]], axis=0)
        yf = jnp.concatenate([ly[:, H:], lvy[:, H:]], axis=0)
        rf = _dot(xf, e0f, _LO) + _dot(yf, e1f, _LO)
        o_gpos[g] = rf[:P]
        o_gvel[g] = rf[P:]

        o_tvalid[g] = g_valid[:, :H] > 0.5
        o_tspd[g] = g_spd[:, :H]
        o_tacc[g] = g_acc[:, :H]
        o_tyaw[g] = lyaw[:, :H]
        o_tyawr[g] = g_yawr[:, :H]
        o_gvalid[g] = g_valid[:, H:] > 0.5
        o_gspd[g] = g_spd[:, H:]
        o_gyaw[g] = lyaw[:, H:]


def kernel(agent_valid, agent_pos, agent_vel, agent_spd, agent_acc,
           agent_yaw_bbox, agent_yaw_rate, agent_type, agent_role,
           agent_size, agent_cmd):
    S, T, A = agent_valid.shape
    P = _N_TARGET
    H = _N_HIST
    F = T - H
    f32 = jnp.float32

    statics = jnp.concatenate([agent_type.astype(f32),
                               agent_role.astype(f32),
                               agent_size, agent_cmd], axis=-1)
    pos = agent_pos.reshape(S, T, 2 * A)
    vel = agent_vel.reshape(S, T, 2 * A)
    spd = agent_spd.reshape(S, T, A)
    acc = agent_acc.reshape(S, T, A)
    yawr = agent_yaw_rate.reshape(S, T, A)
    yaw = agent_yaw_bbox.reshape(S, T, A)

    out_shapes = (
        jax.ShapeDtypeStruct((S, 1, P), jnp.int32),
        jax.ShapeDtypeStruct((S, P, 2), f32),
        jax.ShapeDtypeStruct((S, P, 4), f32),
        jax.ShapeDtypeStruct((S, P, 3), jnp.bool_),
        jax.ShapeDtypeStruct((S, P, 3), jnp.bool_),
        jax.ShapeDtypeStruct((S, P, H), jnp.bool_),
        jax.ShapeDtypeStruct((S, P, 2 * H), f32),
        jax.ShapeDtypeStruct((S, P, 2 * H), f32),
        jax.ShapeDtypeStruct((S, P, H), f32),
        jax.ShapeDtypeStruct((S, P, H), f32),
        jax.ShapeDtypeStruct((S, P, H), f32),
        jax.ShapeDtypeStruct((S, P, H), f32),
        jax.ShapeDtypeStruct((S, P, 3), f32),
        jax.ShapeDtypeStruct((S, P, F), jnp.bool_),
        jax.ShapeDtypeStruct((S, P, 2 * F), f32),
        jax.ShapeDtypeStruct((S, P, F), f32),
        jax.ShapeDtypeStruct((S, P, 2 * F), f32),
        jax.ShapeDtypeStruct((S, P, F), f32),
        jax.ShapeDtypeStruct((S, P, 8), f32),
    )

    G = _SCENES_PER_STEP

    def spec(*dims):
        return pl.BlockSpec((G,) + dims, lambda s: (s,) + (0,) * len(dims))

    outs = pl.pallas_call(
        _group_kernel,
        grid=(S // G,),
        in_specs=[
            spec(T, 2 * A),
            spec(T, 2 * A),
            spec(T, A),
            spec(T, A),
            spec(T, A),
            spec(T, A),
            spec(T, A),
            spec(A, 17),
        ],
        out_specs=tuple(spec(*o.shape[1:]) for o in out_shapes),
        out_shape=out_shapes,
    )(pos, vel, spd, acc, yawr, agent_valid, yaw, statics)

    (o_idx, o_refpos, o_refrot, o_type, o_role, o_tvalid, o_tpos, o_tvel,
     o_tspd, o_tacc, o_tyaw, o_tyawr, o_size, o_gvalid, o_gpos, o_gspd,
     o_gvel, o_gyaw, o_gcmd) = outs

    return (o_idx.reshape(S, P), o_refpos.reshape(S, P, 1, 2),
            o_refrot.reshape(S, P, 2, 2), o_type, o_role,
            o_tvalid, o_tpos.reshape(S, P, H, 2), o_tvel.reshape(S, P, H, 2),
            o_tspd.reshape(S, P, H, 1), o_tacc.reshape(S, P, H, 1),
            o_tyaw.reshape(S, P, H, 1), o_tyawr.reshape(S, P, H, 1),
            o_type, o_role, o_size,
            o_gvalid, o_gpos.reshape(S, P, F, 2),
            o_gspd.reshape(S, P, F, 1), o_gvel.reshape(S, P, F, 2),
            o_gyaw.reshape(S, P, F, 1), o_gcmd)


# speedup vs baseline: 1.0507x; 1.0507x over previous
import jax
import jax.numpy as jnp
from jax.experimental import pallas as pl
from jax.experimental.pallas import tpu as pltpu

_STEP_CURRENT = 10
_N_HIST = _STEP_CURRENT + 1
_N_TARGET = 8
_PI = 3.141592653589793
_HI = jax.lax.Precision.HIGHEST
_LO = jax.lax.Precision.DEFAULT
_SCENES_PER_STEP = 16
_T = 91
_A = 64


def _wrap_rad(x):
    m = x + _PI
    m = m - (2.0 * _PI) * jnp.floor(m / (2.0 * _PI))
    return m - _PI


def _dot_t(a, b, prec):
    return jax.lax.dot_general(
        a, b, (((1,), (1,)), ((), ())), precision=prec,
        preferred_element_type=jnp.float32)


def _dot(a, b, prec):
    return jax.lax.dot_general(
        a, b, (((1,), (0,)), ((), ())), precision=prec,
        preferred_element_type=jnp.float32)


def _group_kernel(pos_ref, vel_ref, spd_ref, acc_ref, yawr_ref, valid_ref,
                  yaw_ref, type_ref, role_ref, size_ref, cmd_ref,
                  o_idx, o_refpos, o_refrot, o_type, o_role,
                  o_tvalid, o_tpos, o_tvel, o_tspd, o_tacc, o_tyaw, o_tyawr,
                  o_size, o_gvalid, o_gpos, o_gspd, o_gvel, o_gyaw, o_gcmd):
    A = _A
    P = _N_TARGET
    T = _T
    H = _N_HIST
    F = T - H
    C = _STEP_CURRENT
    G = _SCENES_PER_STEP

    l_row = jax.lax.broadcasted_iota(jnp.int32, (A, 2 * A), 1)
    a2_col = 2 * jax.lax.broadcasted_iota(jnp.int32, (A, 2 * A), 0)
    d0 = (l_row == a2_col).astype(jnp.float32)
    d1 = (l_row == a2_col + 1).astype(jnp.float32)
    t_row_h = jax.lax.broadcasted_iota(jnp.int32, (H, 2 * H), 1)
    t2_col_h = 2 * jax.lax.broadcasted_iota(jnp.int32, (H, 2 * H), 0)
    e0h = (t_row_h == t2_col_h).astype(jnp.float32)
    e1h = (t_row_h == t2_col_h + 1).astype(jnp.float32)
    t_row_f = jax.lax.broadcasted_iota(jnp.int32, (F, 2 * F), 1)
    t2_col_f = 2 * jax.lax.broadcasted_iota(jnp.int32, (F, 2 * F), 0)
    e0f = (t_row_f == t2_col_f).astype(jnp.float32)
    e1f = (t_row_f == t2_col_f + 1).astype(jnp.float32)
    p_col = jax.lax.broadcasted_iota(jnp.int32, (P, 1), 0)
    a_row = jax.lax.broadcasted_iota(jnp.int32, (P, A), 1)
    neg_a_row1 = A - 1 - jax.lax.broadcasted_iota(jnp.int32, (1, A), 1)

    sel_fs, sel2xs, sel2ys = [], [], []
    for g in range(G):
        role = role_ref[g].astype(jnp.float32)
        w_col = jnp.sum(role, axis=1, keepdims=True)
        w_row = jnp.transpose(w_col) + valid_ref[g, C:C + 1, :].astype(
            jnp.float32)
        key_row = w_row.astype(jnp.int32) * A + neg_a_row1
        key_col = jnp.transpose(key_row)
        rank_col = jnp.sum((key_row > key_col).astype(jnp.int32), axis=1,
                           keepdims=True)
        rank_row = jnp.transpose(rank_col)
        sel = (rank_row == p_col)
        idx_col = jnp.sum(jnp.where(sel, a_row, 0), axis=1, keepdims=True)
        o_idx[g] = jnp.transpose(idx_col)
        sel_f = sel.astype(jnp.float32)
        sel_fs.append(sel_f)
        sel2xs.append(_dot(sel_f, d0, _LO))
        sel2ys.append(_dot(sel_f, d1, _LO))

    gath = []
    for g in range(G):
        px = _dot_t(sel2xs[g], pos_ref[g], _HI)
        py = _dot_t(sel2ys[g], pos_ref[g], _HI)
        vx = _dot_t(sel2xs[g], vel_ref[g], _HI)
        vy = _dot_t(sel2ys[g], vel_ref[g], _HI)
        g_spd = _dot_t(sel_fs[g], spd_ref[g], _LO)
        g_acc = _dot_t(sel_fs[g], acc_ref[g], _LO)
        g_yawr = _dot_t(sel_fs[g], yawr_ref[g], _LO)
        g_valid = _dot_t(sel_fs[g], valid_ref[g].astype(jnp.float32), _LO)
        g_yaw = _dot_t(sel_fs[g], yaw_ref[g], _HI)
        g_type = _dot(sel_fs[g], type_ref[g].astype(jnp.float32), _LO)
        g_role = _dot(sel_fs[g], role_ref[g].astype(jnp.float32), _LO)
        g_size = _dot(sel_fs[g], size_ref[g], _LO)
        g_cmd = _dot(sel_fs[g], cmd_ref[g], _LO)
        gath.append((px, py, vx, vy, g_spd, g_acc, g_yawr, g_valid,
                     g_yaw, g_type, g_role, g_size, g_cmd))

    for g in range(G):
        (px, py, vx, vy, g_spd, g_acc, g_yawr, g_valid, g_yaw,
         g_type, g_role, g_size, g_cmd) = gath[g]

        px0 = px[:, C:C + 1]
        py0 = py[:, C:C + 1]
        yaw0 = g_yaw[:, C:C + 1]
        c = jnp.cos(yaw0)
        s = jnp.sin(yaw0)

        dx = px - px0
        dy = py - py0
        lx = dx * c + dy * s
        ly = dy * c - dx * s
        lvx = vx * c + vy * s
        lvy = vy * c - vx * s
        lyaw = _wrap_rad(g_yaw - yaw0)

        o_refpos[g] = jnp.concatenate([px0, py0], axis=1)
        o_refrot[g] = jnp.concatenate([c, -s, s, c], axis=1)
        o_type[g] = g_type > 0.5
        o_role[g] = g_role > 0.5
        o_size[g] = g_size
        o_gcmd[g] = g_cmd

        xh = jnp.concatenate([lx[:, :H], lvx[:, :H]], axis=0)
        yh = jnp.concatenate([ly[:, :H], lvy[:, :H]], axis=0)
        rh = _dot(xh, e0h, _LO) + _dot(yh, e1h, _LO)
        o_tpos[g] = rh[:P]
        o_tvel[g] = rh[P:]
        xf = jnp.concatenate([lx[:, H:], lvx[:, H:]], axis=0)
        yf = jnp.concatenate([ly[:, H:], lvy[:, H:]], axis=0)
        rf = _dot(xf, e0f, _LO) + _dot(yf, e1f, _LO)
        o_gpos[g] = rf[:P]
        o_gvel[g] = rf[P:]

        o_tvalid[g] = g_valid[:, :H] > 0.5
        o_tspd[g] = g_spd[:, :H]
        o_tacc[g] = g_acc[:, :H]
        o_tyaw[g] = lyaw[:, :H]
        o_tyawr[g] = g_yawr[:, :H]
        o_gvalid[g] = g_valid[:, H:] > 0.5
        o_gspd[g] = g_spd[:, H:]
        o_gyaw[g] = lyaw[:, H:]


def kernel(agent_valid, agent_pos, agent_vel, agent_spd, agent_acc,
           agent_yaw_bbox, agent_yaw_rate, agent_type, agent_role,
           agent_size, agent_cmd):
    S, T, A = agent_valid.shape
    P = _N_TARGET
    H = _N_HIST
    F = T - H
    f32 = jnp.float32

    pos = agent_pos.reshape(S, T, 2 * A)
    vel = agent_vel.reshape(S, T, 2 * A)
    spd = agent_spd.reshape(S, T, A)
    acc = agent_acc.reshape(S, T, A)
    yawr = agent_yaw_rate.reshape(S, T, A)
    yaw = agent_yaw_bbox.reshape(S, T, A)

    out_shapes = (
        jax.ShapeDtypeStruct((S, 1, P), jnp.int32),
        jax.ShapeDtypeStruct((S, P, 2), f32),
        jax.ShapeDtypeStruct((S, P, 4), f32),
        jax.ShapeDtypeStruct((S, P, 3), jnp.bool_),
        jax.ShapeDtypeStruct((S, P, 3), jnp.bool_),
        jax.ShapeDtypeStruct((S, P, H), jnp.bool_),
        jax.ShapeDtypeStruct((S, P, 2 * H), f32),
        jax.ShapeDtypeStruct((S, P, 2 * H), f32),
        jax.ShapeDtypeStruct((S, P, H), f32),
        jax.ShapeDtypeStruct((S, P, H), f32),
        jax.ShapeDtypeStruct((S, P, H), f32),
        jax.ShapeDtypeStruct((S, P, H), f32),
        jax.ShapeDtypeStruct((S, P, 3), f32),
        jax.ShapeDtypeStruct((S, P, F), jnp.bool_),
        jax.ShapeDtypeStruct((S, P, 2 * F), f32),
        jax.ShapeDtypeStruct((S, P, F), f32),
        jax.ShapeDtypeStruct((S, P, 2 * F), f32),
        jax.ShapeDtypeStruct((S, P, F), f32),
        jax.ShapeDtypeStruct((S, P, 8), f32),
    )

    G = _SCENES_PER_STEP

    def spec(*dims):
        return pl.BlockSpec((G,) + dims, lambda s: (s,) + (0,) * len(dims))

    outs = pl.pallas_call(
        _group_kernel,
        grid=(S // G,),
        in_specs=[
            spec(T, 2 * A),
            spec(T, 2 * A),
            spec(T, A),
            spec(T, A),
            spec(T, A),
            spec(T, A),
            spec(T, A),
            spec(A, 3),
            spec(A, 3),
            spec(A, 3),
            spec(A, 8),
        ],
        out_specs=tuple(spec(*o.shape[1:]) for o in out_shapes),
        out_shape=out_shapes,
    )(pos, vel, spd, acc, yawr, agent_valid, yaw,
      agent_type, agent_role, agent_size, agent_cmd)

    (o_idx, o_refpos, o_refrot, o_type, o_role, o_tvalid, o_tpos, o_tvel,
     o_tspd, o_tacc, o_tyaw, o_tyawr, o_size, o_gvalid, o_gpos, o_gspd,
     o_gvel, o_gyaw, o_gcmd) = outs

    return (o_idx.reshape(S, P), o_refpos.reshape(S, P, 1, 2),
            o_refrot.reshape(S, P, 2, 2), o_type, o_role,
            o_tvalid, o_tpos.reshape(S, P, H, 2), o_tvel.reshape(S, P, H, 2),
            o_tspd.reshape(S, P, H, 1), o_tacc.reshape(S, P, H, 1),
            o_tyaw.reshape(S, P, H, 1), o_tyawr.reshape(S, P, H, 1),
            o_type, o_role, o_size,
            o_gvalid, o_gpos.reshape(S, P, F, 2),
            o_gspd.reshape(S, P, F, 1), o_gvel.reshape(S, P, F, 2),
            o_gyaw.reshape(S, P, F, 1), o_gcmd)


# speedup vs baseline: 1.1908x; 1.1333x over previous
import jax
import jax.numpy as jnp
from jax.experimental import pallas as pl
from jax.experimental.pallas import tpu as pltpu

_STEP_CURRENT = 10
_N_HIST = _STEP_CURRENT + 1
_N_TARGET = 8
_PI = 3.141592653589793
_HI = jax.lax.Precision.HIGHEST
_LO = jax.lax.Precision.DEFAULT
_SCENES_PER_STEP = 16
_T = 91
_A = 64


def _wrap_rad(x):
    m = x + _PI
    m = m - (2.0 * _PI) * jnp.floor(m / (2.0 * _PI))
    return m - _PI


def _dot_t(a, b, prec):
    return jax.lax.dot_general(
        a, b, (((1,), (1,)), ((), ())), precision=prec,
        preferred_element_type=jnp.float32)


def _dot(a, b, prec):
    return jax.lax.dot_general(
        a, b, (((1,), (0,)), ((), ())), precision=prec,
        preferred_element_type=jnp.float32)


def _group_kernel(pos_ref, vel_ref, spd_ref, acc_ref, yawr_ref, valid_ref,
                  yaw_ref, type_ref, role_ref, size_ref, cmd_ref,
                  o_idx, o_refpos, o_refrot, o_type, o_role,
                  o_tvalid, o_tpos, o_tvel, o_tspd, o_tacc, o_tyaw, o_tyawr,
                  o_size, o_gvalid, o_gpos, o_gspd, o_gvel, o_gyaw, o_gcmd):
    A = _A
    P = _N_TARGET
    T = _T
    H = _N_HIST
    F = T - H
    C = _STEP_CURRENT
    G = _SCENES_PER_STEP

    l_row = jax.lax.broadcasted_iota(jnp.int32, (A, 2 * A), 1)
    a2_col = 2 * jax.lax.broadcasted_iota(jnp.int32, (A, 2 * A), 0)
    d0 = (l_row == a2_col).astype(jnp.float32)
    d1 = (l_row == a2_col + 1).astype(jnp.float32)
    t_row_h = jax.lax.broadcasted_iota(jnp.int32, (H, 2 * H), 1)
    t2_col_h = 2 * jax.lax.broadcasted_iota(jnp.int32, (H, 2 * H), 0)
    e0h = (t_row_h == t2_col_h).astype(jnp.float32)
    e1h = (t_row_h == t2_col_h + 1).astype(jnp.float32)
    t_row_f = jax.lax.broadcasted_iota(jnp.int32, (F, 2 * F), 1)
    t2_col_f = 2 * jax.lax.broadcasted_iota(jnp.int32, (F, 2 * F), 0)
    e0f = (t_row_f == t2_col_f).astype(jnp.float32)
    e1f = (t_row_f == t2_col_f + 1).astype(jnp.float32)
    p_col = jax.lax.broadcasted_iota(jnp.int32, (P, 1), 0)
    a_row = jax.lax.broadcasted_iota(jnp.int32, (P, A), 1)
    neg_a_row1 = A - 1 - jax.lax.broadcasted_iota(jnp.int32, (1, A), 1)

    sel_fs, sel2xs, sel2ys = [], [], []
    for g in range(G):
        role = role_ref[g].astype(jnp.float32)
        w_col = jnp.sum(role, axis=1, keepdims=True)
        w_row = jnp.transpose(w_col) + valid_ref[g, C:C + 1, :].astype(
            jnp.float32)
        key_row = w_row.astype(jnp.int32) * A + neg_a_row1
        key_col = jnp.transpose(key_row)
        rank_col = jnp.sum((key_row > key_col).astype(jnp.int32), axis=1,
                           keepdims=True)
        rank_row = jnp.transpose(rank_col)
        sel = (rank_row == p_col)
        idx_col = jnp.sum(jnp.where(sel, a_row, 0), axis=1, keepdims=True)
        o_idx[g] = jnp.transpose(idx_col)
        sel_f = sel.astype(jnp.float32)
        sel_fs.append(sel_f)
        sel2xs.append(_dot(sel_f, d0, _LO))
        sel2ys.append(_dot(sel_f, d1, _LO))

    gath = []
    for g in range(G):
        px = _dot_t(sel2xs[g], pos_ref[g], _LO)
        py = _dot_t(sel2ys[g], pos_ref[g], _LO)
        vx = _dot_t(sel2xs[g], vel_ref[g], _LO)
        vy = _dot_t(sel2ys[g], vel_ref[g], _LO)
        g_spd = _dot_t(sel_fs[g], spd_ref[g], _LO)
        g_acc = _dot_t(sel_fs[g], acc_ref[g], _LO)
        g_yawr = _dot_t(sel_fs[g], yawr_ref[g], _LO)
        g_valid = _dot_t(sel_fs[g], valid_ref[g].astype(jnp.float32), _LO)
        g_yaw = _dot_t(sel_fs[g], yaw_ref[g], _HI)
        g_type = _dot(sel_fs[g], type_ref[g].astype(jnp.float32), _LO)
        g_role = _dot(sel_fs[g], role_ref[g].astype(jnp.float32), _LO)
        g_size = _dot(sel_fs[g], size_ref[g], _LO)
        g_cmd = _dot(sel_fs[g], cmd_ref[g], _LO)
        gath.append((px, py, vx, vy, g_spd, g_acc, g_yawr, g_valid,
                     g_yaw, g_type, g_role, g_size, g_cmd))

    for g in range(G):
        (px, py, vx, vy, g_spd, g_acc, g_yawr, g_valid, g_yaw,
         g_type, g_role, g_size, g_cmd) = gath[g]

        px0 = px[:, C:C + 1]
        py0 = py[:, C:C + 1]
        yaw0 = g_yaw[:, C:C + 1]
        c = jnp.cos(yaw0)
        s = jnp.sin(yaw0)

        dx = px - px0
        dy = py - py0
        lx = dx * c + dy * s
        ly = dy * c - dx * s
        lvx = vx * c + vy * s
        lvy = vy * c - vx * s
        lyaw = _wrap_rad(g_yaw - yaw0)

        o_refpos[g] = jnp.concatenate([px0, py0], axis=1)
        o_refrot[g] = jnp.concatenate([c, -s, s, c], axis=1)
        o_type[g] = g_type > 0.5
        o_role[g] = g_role > 0.5
        o_size[g] = g_size
        o_gcmd[g] = g_cmd

        xh = jnp.concatenate([lx[:, :H], lvx[:, :H]], axis=0)
        yh = jnp.concatenate([ly[:, :H], lvy[:, :H]], axis=0)
        rh = _dot(xh, e0h, _LO) + _dot(yh, e1h, _LO)
        o_tpos[g] = rh[:P]
        o_tvel[g] = rh[P:]
        xf = jnp.concatenate([lx[:, H:], lvx[:, H:]], axis=0)
        yf = jnp.concatenate([ly[:, H:], lvy[:, H:]], axis=0)
        rf = _dot(xf, e0f, _LO) + _dot(yf, e1f, _LO)
        o_gpos[g] = rf[:P]
        o_gvel[g] = rf[P:]

        o_tvalid[g] = g_valid[:, :H] > 0.5
        o_tspd[g] = g_spd[:, :H]
        o_tacc[g] = g_acc[:, :H]
        o_tyaw[g] = lyaw[:, :H]
        o_tyawr[g] = g_yawr[:, :H]
        o_gvalid[g] = g_valid[:, H:] > 0.5
        o_gspd[g] = g_spd[:, H:]
        o_gyaw[g] = lyaw[:, H:]


def kernel(agent_valid, agent_pos, agent_vel, agent_spd, agent_acc,
           agent_yaw_bbox, agent_yaw_rate, agent_type, agent_role,
           agent_size, agent_cmd):
    S, T, A = agent_valid.shape
    P = _N_TARGET
    H = _N_HIST
    F = T - H
    f32 = jnp.float32

    pos = agent_pos.reshape(S, T, 2 * A)
    vel = agent_vel.reshape(S, T, 2 * A)
    spd = agent_spd.reshape(S, T, A)
    acc = agent_acc.reshape(S, T, A)
    yawr = agent_yaw_rate.reshape(S, T, A)
    yaw = agent_yaw_bbox.reshape(S, T, A)

    out_shapes = (
        jax.ShapeDtypeStruct((S, 1, P), jnp.int32),
        jax.ShapeDtypeStruct((S, P, 2), f32),
        jax.ShapeDtypeStruct((S, P, 4), f32),
        jax.ShapeDtypeStruct((S, P, 3), jnp.bool_),
        jax.ShapeDtypeStruct((S, P, 3), jnp.bool_),
        jax.ShapeDtypeStruct((S, P, H), jnp.bool_),
        jax.ShapeDtypeStruct((S, P, 2 * H), f32),
        jax.ShapeDtypeStruct((S, P, 2 * H), f32),
        jax.ShapeDtypeStruct((S, P, H), f32),
        jax.ShapeDtypeStruct((S, P, H), f32),
        jax.ShapeDtypeStruct((S, P, H), f32),
        jax.ShapeDtypeStruct((S, P, H), f32),
        jax.ShapeDtypeStruct((S, P, 3), f32),
        jax.ShapeDtypeStruct((S, P, F), jnp.bool_),
        jax.ShapeDtypeStruct((S, P, 2 * F), f32),
        jax.ShapeDtypeStruct((S, P, F), f32),
        jax.ShapeDtypeStruct((S, P, 2 * F), f32),
        jax.ShapeDtypeStruct((S, P, F), f32),
        jax.ShapeDtypeStruct((S, P, 8), f32),
    )

    G = _SCENES_PER_STEP

    def spec(*dims):
        return pl.BlockSpec((G,) + dims, lambda s: (s,) + (0,) * len(dims))

    outs = pl.pallas_call(
        _group_kernel,
        grid=(S // G,),
        in_specs=[
            spec(T, 2 * A),
            spec(T, 2 * A),
            spec(T, A),
            spec(T, A),
            spec(T, A),
            spec(T, A),
            spec(T, A),
            spec(A, 3),
            spec(A, 3),
            spec(A, 3),
            spec(A, 8),
        ],
        out_specs=tuple(spec(*o.shape[1:]) for o in out_shapes),
        out_shape=out_shapes,
    )(pos, vel, spd, acc, yawr, agent_valid, yaw,
      agent_type, agent_role, agent_size, agent_cmd)

    (o_idx, o_refpos, o_refrot, o_type, o_role, o_tvalid, o_tpos, o_tvel,
     o_tspd, o_tacc, o_tyaw, o_tyawr, o_size, o_gvalid, o_gpos, o_gspd,
     o_gvel, o_gyaw, o_gcmd) = outs

    return (o_idx.reshape(S, P), o_refpos.reshape(S, P, 1, 2),
            o_refrot.reshape(S, P, 2, 2), o_type, o_role,
            o_tvalid, o_tpos.reshape(S, P, H, 2), o_tvel.reshape(S, P, H, 2),
            o_tspd.reshape(S, P, H, 1), o_tacc.reshape(S, P, H, 1),
            o_tyaw.reshape(S, P, H, 1), o_tyawr.reshape(S, P, H, 1),
            o_type, o_role, o_size,
            o_gvalid, o_gpos.reshape(S, P, F, 2),
            o_gspd.reshape(S, P, F, 1), o_gvel.reshape(S, P, F, 2),
            o_gyaw.reshape(S, P, F, 1), o_gcmd)
